# baseline (speedup 1.0000x reference)
import functools

import jax
import jax.numpy as jnp
from jax import lax
from jax.experimental import pallas as pl
from jax.experimental.pallas import tpu as pltpu
from jax.experimental.pallas import tpu_sc as plsc

N = 10000
NPAD = 10240
E = 160000
HID = 512
FB = 4
FBW = 128
FC = 1024
C = 10
G = 64
ED = 4
L = 3

NC = 2
NS = 16
NW = NC * NS
CH = 128
NCHUNKS = E // CH
TMAX = (NCHUNKS + NW - 1) // NW

_mesh = plsc.VectorSubcoreMesh(core_axis_name="c", subcore_axis_name="s")


def _mm_fb(a, w, b2d, bm):
    m, k = a.shape
    nb = w.shape[1] // FBW
    grid = (m // bm, nb)

    def body(a_ref, w_ref, b_ref, o_ref):
        o_ref[...] = (
            jnp.dot(a_ref[...], w_ref[...], preferred_element_type=jnp.float32)
            + b_ref[0:1]
        )

    return pl.pallas_call(
        body,
        grid=grid,
        in_specs=[
            pl.BlockSpec((bm, k), lambda i, f: (i, 0)),
            pl.BlockSpec((k, FBW), lambda i, f: (0, f)),
            pl.BlockSpec((8, FBW), lambda i, f: (f, 0)),
        ],
        out_specs=pl.BlockSpec((bm, FBW), lambda i, f: (f * (m // bm) + i, 0)),
        out_shape=jax.ShapeDtypeStruct((nb * m, FBW), jnp.float32),
    )(a, w, b2d)


def _make_sc_pass1(lofs):
    @functools.partial(
        pl.kernel,
        out_type=jax.ShapeDtypeStruct((E * 16,), jnp.float32),
        mesh=_mesh,
        scratch_types=[
            pltpu.VMEM((CH,), jnp.int32),
            pltpu.VMEM((CH,), jnp.int32),
            pltpu.VMEM((CH,), jnp.int32),
            pltpu.VMEM((CH,), jnp.int32),
            pltpu.VMEM((CH, FBW), jnp.float32),
            pltpu.VMEM((CH, FBW), jnp.float32),
            pltpu.VMEM((CH, FBW), jnp.float32),
            pltpu.VMEM((FB, FBW), jnp.float32),
            pltpu.VMEM((CH * 16,), jnp.float32),
            pltpu.SemaphoreType.DMA,
            pltpu.SemaphoreType.DMA,
        ],
    )
    def _sc_pass1(xlxr_hbm, ee_hbm, src_hbm, dst_hbm, att_hbm,
                  part_out,
                  srcv, dstv, sidx, didx, xlr, xrr, eer, attv,
                  alanes, sem1, sem2):
        wid = lax.axis_index("s") * NC + lax.axis_index("c")

        pltpu.sync_copy(att_hbm, attv)

        def chunk_body(t, _):
            cid = wid + t * NW

            @pl.when(cid < NCHUNKS)
            def _():
                eb = cid * CH
                pltpu.sync_copy(src_hbm.at[pl.ds(eb, CH)], srcv)
                pltpu.sync_copy(dst_hbm.at[pl.ds(eb, CH)], dstv)

                for fb in range(FB):
                    off = fb * NPAD
                    offr = (FB + fb) * NPAD
                    for g in range(CH // 16):
                        sl = pl.ds(g * 16, 16)
                        sidx[sl] = srcv[sl] + off
                        didx[sl] = dstv[sl] + offr
                    cp1 = pltpu.async_copy(xlxr_hbm.at[sidx], xlr, sem1)
                    cp2 = pltpu.async_copy(xlxr_hbm.at[didx], xrr, sem2)
                    cp1.wait()
                    cp2.wait()
                    pltpu.sync_copy(
                        ee_hbm.at[pl.ds((lofs + fb) * E + eb, CH)], eer)

                    att_vs = [attv[fb, pl.ds(j * 16, 16)]
                              for j in range(FBW // 16)]

                    def edge_body(i, _, fb=fb, att_vs=att_vs):
                        acc = None
                        for j in range(FBW // 16):
                            sl = pl.ds(j * 16, 16)
                            mm = xlr[i, sl] + xrr[i, sl] + eer[i, sl]
                            mm = jnp.maximum(mm, 0.2 * mm)
                            contrib = mm * att_vs[j]
                            acc = contrib if acc is None else acc + contrib
                        osl = pl.ds(i * 16, 16)
                        if fb == 0:
                            alanes[osl] = acc
                        else:
                            alanes[osl] = alanes[osl] + acc
                        return 0
                    lax.fori_loop(0, CH, edge_body, 0)

                pltpu.sync_copy(alanes, part_out.at[pl.ds(eb * 16, CH * 16)])
            return 0

        lax.fori_loop(0, TMAX, chunk_body, 0)

    return _sc_pass1


def _tc_edge_w(part, dmat):
    rows = E * 16 // FBW
    bm = 800
    grid = (rows // bm,)

    def body(p_ref, d_ref, o_ref):
        a = jnp.dot(p_ref[...], d_ref[...], preferred_element_type=jnp.float32)
        o_ref[...] = jnp.exp(a)

    return pl.pallas_call(
        body,
        grid=grid,
        in_specs=[
            pl.BlockSpec((bm, FBW), lambda i: (i, 0)),
            pl.BlockSpec((FBW, FBW), lambda i: (0, 0)),
        ],
        out_specs=pl.BlockSpec((bm, FBW), lambda i: (i, 0)),
        out_shape=jax.ShapeDtypeStruct((rows, FBW), jnp.float32),
    )(part.reshape(rows, FBW), dmat)


@functools.partial(
    pl.kernel,
    out_type=[
        jax.ShapeDtypeStruct((NC, FB, NPAD, FBW), jnp.float32),
        jax.ShapeDtypeStruct((NC, NPAD), jnp.float32),
    ],
    mesh=_mesh,
    scratch_types=[
        pltpu.VMEM((2, CH), jnp.int32),
        pltpu.VMEM((2, CH), jnp.int32),
        pltpu.VMEM((2, CH), jnp.int32),
        pltpu.VMEM((2, CH * 16 // FBW, FBW), jnp.float32),
        pltpu.VMEM((CH,), jnp.float32),
        pltpu.VMEM((2, CH, FBW), jnp.float32),
        pltpu.VMEM((64, FBW), jnp.float32),
        pltpu.VMEM_SHARED((NPAD, FBW), jnp.float32),
        pltpu.VMEM_SHARED((NPAD,), jnp.float32),
        pltpu.SemaphoreType.DMA,
        pltpu.SemaphoreType.DMA,
        pltpu.SemaphoreType.DMA,
        pltpu.SemaphoreType.DMA,
    ],
)
def _sc_pass2(xl_hbm, src_hbm, dst_hbm, w_hbm,
              out_hbm, dpart_out,
              srcv2, dstv2, sidx2, wb2, wbufc, xlr2, zrow, acc, densh,
              sems0, sems1, semg0, semg1):
    cidx = lax.axis_index("c")
    sid = lax.axis_index("s")
    wid = sid * NC + cidx
    semss = [sems0, sems1]
    semgs = [semg0, semg1]

    def zrow_body(i, _):
        for j in range(FBW // 16):
            zrow[i, pl.ds(j * 16, 16)] = jnp.zeros((16,), jnp.float32)
        return 0
    lax.fori_loop(0, 64, zrow_body, 0)

    rows_per_sub = NPAD // NS

    for fb in range(FB):
        for k in range(rows_per_sub // 64):
            pltpu.sync_copy(zrow, acc.at[pl.ds(sid * rows_per_sub + k * 64, 64)])
        if fb == 0:
            for k in range(rows_per_sub // FBW):
                pltpu.sync_copy(
                    zrow.at[0],
                    densh.at[pl.ds(sid * rows_per_sub + k * FBW, FBW)])
        plsc.subcore_barrier()

        def fire(t, b, fb=fb):
            cid = wid + t * NW

            @pl.when(cid < NCHUNKS)
            def _():
                eb = cid * CH
                c1 = pltpu.async_copy(src_hbm.at[pl.ds(eb, CH)],
                                      srcv2.at[b], semss[b])
                c2 = pltpu.async_copy(dst_hbm.at[pl.ds(eb, CH)],
                                      dstv2.at[b], semss[b])
                c3 = pltpu.async_copy(
                    w_hbm.at[pl.ds(cid * (CH * 16 // FBW), CH * 16 // FBW)],
                    wb2.at[b], semss[b])
                c1.wait()
                c2.wait()
                c3.wait()
                off = fb * NPAD
                for g in range(CH // 16):
                    sl = pl.ds(g * 16, 16)
                    sidx2[b, sl] = srcv2[b, sl] + off
                pltpu.async_copy(xl_hbm.at[sidx2.at[b]], xlr2.at[b], semgs[b])

        def consume(t, b, fb=fb):
            cid = wid + t * NW

            @pl.when(cid < NCHUNKS)
            def _():
                pltpu.make_async_copy(xl_hbm.at[sidx2.at[b]], xlr2.at[b],
                                      semgs[b]).wait()

                if fb == 0:
                    iv16 = lax.iota(jnp.int32, 16)
                    for g in range(CH // 16):
                        wc = jnp.zeros((16,), jnp.float32)
                        for k in range(16):
                            v = wb2[b, 2 * g + k // 8,
                                    pl.ds((k % 8) * 16, 16)]
                            wc = jnp.where(iv16 == k, v[0], wc)
                        wbufc[pl.ds(g * 16, 16)] = wc
                    pltpu.sync_copy(wbufc, densh.at[dstv2.at[b]], add=True)

                def scale_body(g, _):
                    for k in range(16):
                        i = g * 16 + k
                        wsv = wb2[b, 2 * g + k // 8, pl.ds((k % 8) * 16, 16)]
                        for j in range(FBW // 16):
                            sl = pl.ds(j * 16, 16)
                            xlr2[b, i, sl] = xlr2[b, i, sl] * wsv
                    return 0
                lax.fori_loop(0, CH // 16, scale_body, 0)

                pltpu.sync_copy(xlr2.at[b], acc.at[dstv2.at[b]], add=True)

        fire(0, 0)

        def super_body(s, _):
            k0 = s * 2
            fire(k0 + 1, 1)
            consume(k0, 0)
            fire(k0 + 2, 0)
            consume(k0 + 1, 1)
            return 0

        lax.fori_loop(0, TMAX // 2, super_body, 0)
        plsc.subcore_barrier()

        for k in range(rows_per_sub // 64):
            base = sid * rows_per_sub + k * 64
            pltpu.sync_copy(acc.at[pl.ds(base, 64)],
                            out_hbm.at[cidx, fb, pl.ds(base, 64)])
        if fb == 0:
            pltpu.sync_copy(densh.at[pl.ds(sid * rows_per_sub, rows_per_sub)],
                            dpart_out.at[cidx, pl.ds(sid * rows_per_sub,
                                                     rows_per_sub)])
        plsc.subcore_barrier()


def _tc_combine(outp, dpart, bias2d):
    bn = 1024
    grid = (NPAD // bn, FB)

    def body(o_ref, d_ref, b_ref, h_ref):
        s = o_ref[0, 0] + o_ref[1, 0]
        den = jnp.sum(d_ref[...], axis=0)
        x = s / (den[:, None] + 1e-16) + b_ref[0:1]
        h_ref[...] = jnp.where(x > 0.0, x, jnp.exp(jnp.minimum(x, 0.0)) - 1.0)

    return pl.pallas_call(
        body,
        grid=grid,
        in_specs=[
            pl.BlockSpec((NC, 1, bn, FBW), lambda i, f: (0, f, i, 0)),
            pl.BlockSpec((NC, bn), lambda i, f: (0, i)),
            pl.BlockSpec((8, FBW), lambda i, f: (f, 0)),
        ],
        out_specs=pl.BlockSpec((bn, FBW), lambda i, f: (i, f)),
        out_shape=jax.ShapeDtypeStruct((NPAD, HID), jnp.float32),
    )(outp, dpart, bias2d)


def _tc_combine_mm(outp, dpart, bias2d, wcat, bcat2d):
    bm = 512
    nb = wcat.shape[1] // FBW
    grid = (NPAD // bm, nb)

    def body(o_ref, d_ref, b_ref, w_ref, bc_ref, x_ref, h_scr):
        f = pl.program_id(1)

        @pl.when(f == 0)
        def _():
            den = jnp.sum(d_ref[...], axis=0)[:, None]
            for fb in range(FB):
                s = o_ref[0, fb] + o_ref[1, fb]
                x = s / (den + 1e-16) + b_ref[fb * 8:fb * 8 + 1]
                h_scr[:, fb * FBW:(fb + 1) * FBW] = jnp.where(
                    x > 0.0, x, jnp.exp(jnp.minimum(x, 0.0)) - 1.0)

        x_ref[...] = (
            jnp.dot(h_scr[...], w_ref[...], preferred_element_type=jnp.float32)
            + bc_ref[0:1]
        )

    return pl.pallas_call(
        body,
        grid=grid,
        in_specs=[
            pl.BlockSpec((NC, FB, bm, FBW), lambda i, f: (0, 0, i, 0)),
            pl.BlockSpec((NC, bm), lambda i, f: (0, i)),
            pl.BlockSpec((FB * 8, FBW), lambda i, f: (0, 0)),
            pl.BlockSpec((HID, FBW), lambda i, f: (0, f)),
            pl.BlockSpec((8, FBW), lambda i, f: (f, 0)),
        ],
        out_specs=pl.BlockSpec((bm, FBW),
                               lambda i, f: (f * (NPAD // bm) + i, 0)),
        out_shape=jax.ShapeDtypeStruct((nb * NPAD, FBW), jnp.float32),
        scratch_shapes=[pltpu.VMEM((bm, HID), jnp.float32)],
    )(outp, dpart, bias2d, wcat, bcat2d)


def _tc_pool_mlp(h, batch2d, wf1, bf1, wf2, bf2):
    bp = 1024
    nsteps = NPAD // bp

    def body(h_ref, b_ref, w1_ref, b1_ref, w2_ref, b2_ref, o_ref,
             pooled, cnt):
        i = pl.program_id(0)

        @pl.when(i == 0)
        def _():
            pooled[...] = jnp.zeros_like(pooled)
            cnt[...] = jnp.zeros_like(cnt)

        oh = (b_ref[...] == lax.broadcasted_iota(jnp.int32, (bp, G), 1)
              ).astype(jnp.float32)
        pooled[...] += lax.dot_general(
            oh, h_ref[...], (((0,), (0,)), ((), ())),
            preferred_element_type=jnp.float32)
        cnt[...] += jnp.sum(oh, axis=0)[:, None]

        @pl.when(i == nsteps - 1)
        def _():
            c = jnp.maximum(cnt[:, 0:1], 1.0)
            pm = pooled[...] / c
            z1 = jnp.dot(pm, w1_ref[...], preferred_element_type=jnp.float32)
            z1 = jnp.maximum(z1 + b1_ref[...], 0.0)
            z = jnp.dot(z1, w2_ref[...], preferred_element_type=jnp.float32)
            z = z + b2_ref[...]
            mz = jnp.max(z, axis=1, keepdims=True)
            sz = z - mz
            o_ref[...] = sz - jnp.log(jnp.sum(jnp.exp(sz), axis=1,
                                              keepdims=True))

    return pl.pallas_call(
        body,
        grid=(nsteps,),
        in_specs=[
            pl.BlockSpec((bp, HID), lambda i: (i, 0)),
            pl.BlockSpec((bp, 1), lambda i: (i, 0)),
            pl.BlockSpec((HID, FC), lambda i: (0, 0)),
            pl.BlockSpec((1, FC), lambda i: (0, 0)),
            pl.BlockSpec((FC, C), lambda i: (0, 0)),
            pl.BlockSpec((1, C), lambda i: (0, 0)),
        ],
        out_specs=pl.BlockSpec((G, C), lambda i: (0, 0)),
        out_shape=jax.ShapeDtypeStruct((G, C), jnp.float32),
        scratch_shapes=[
            pltpu.VMEM((G, HID), jnp.float32),
            pltpu.VMEM((G, FBW), jnp.float32),
        ],
    )(h, batch2d, wf1, bf1, wf2, bf2)


def _rep_bias(b):
    nb = b.shape[0] // FBW
    return jnp.broadcast_to(b.reshape(nb, 1, FBW), (nb, 8, FBW)).reshape(
        nb * 8, FBW)


def kernel(x, edge_index, edge_attr, batch, params):
    src = edge_index[0]
    dst = edge_index[1]

    h = jnp.pad(x, ((0, NPAD - N), (0, 0)))
    batch2d = jnp.concatenate(
        [batch, jnp.full((NPAD - N,), G, jnp.int32)]).reshape(NPAD, 1)
    ii = jnp.arange(FBW, dtype=jnp.int32)
    dmat = (ii[:, None] // 16 == ii[None, :] // 16).astype(jnp.float32)

    we_all = jnp.concatenate([params[l][4] for l in range(L)], axis=1)
    ee_all = _mm_fb(edge_attr, we_all,
                    jnp.zeros((L * FB * 8, FBW), jnp.float32), 1600)

    wl, bl, wr, br = params[0][:4]
    xlxr = _mm_fb(h, jnp.concatenate([wl, wr], axis=1),
                  _rep_bias(jnp.concatenate([bl, br])), 512)

    for l in range(L):
        att, bias = params[l][5], params[l][6]
        part = _make_sc_pass1(l * FB)(xlxr, ee_all, src, dst,
                                      att.reshape(FB, FBW))
        w2d = _tc_edge_w(part, dmat)
        outp, dpart = _sc_pass2(xlxr, src, dst, w2d)
        if l < L - 1:
            wln, bln, wrn, brn = params[l + 1][:4]
            xlxr = _tc_combine_mm(
                outp, dpart, _rep_bias(bias),
                jnp.concatenate([wln, wrn], axis=1),
                _rep_bias(jnp.concatenate([bln, brn])))
        else:
            h = _tc_combine(outp, dpart, _rep_bias(bias))

    wf1, bf1, wf2, bf2 = params[L]
    return _tc_pool_mlp(h, batch2d, wf1, bf1.reshape(1, FC),
                        wf2, bf2.reshape(1, C))


# speedup vs baseline: 1.1165x; 1.1165x over previous
import functools

import jax
import jax.numpy as jnp
from jax import lax
from jax.experimental import pallas as pl
from jax.experimental.pallas import tpu as pltpu
from jax.experimental.pallas import tpu_sc as plsc

N = 10000
NPAD = 10240
E = 160000
HID = 512
FB = 4
FBW = 128
FC = 1024
C = 10
G = 64
ED = 4
L = 3

NC = 2
NS = 16
NW = NC * NS
CH = 128
NCHUNKS = E // CH
TMAX = (NCHUNKS + NW - 1) // NW

_mesh = plsc.VectorSubcoreMesh(core_axis_name="c", subcore_axis_name="s")


def _mm_fb(a, w, b2d, bm):
    m, k = a.shape
    nb = w.shape[1] // FBW
    grid = (m // bm, nb)

    def body(a_ref, w_ref, b_ref, o_ref):
        o_ref[...] = (
            jnp.dot(a_ref[...], w_ref[...], preferred_element_type=jnp.float32)
            + b_ref[0:1]
        )

    return pl.pallas_call(
        body,
        grid=grid,
        in_specs=[
            pl.BlockSpec((bm, k), lambda i, f: (i, 0)),
            pl.BlockSpec((k, FBW), lambda i, f: (0, f)),
            pl.BlockSpec((8, FBW), lambda i, f: (f, 0)),
        ],
        out_specs=pl.BlockSpec((bm, FBW), lambda i, f: (f * (m // bm) + i, 0)),
        out_shape=jax.ShapeDtypeStruct((nb * m, FBW), jnp.float32),
    )(a, w, b2d)


def _make_sc_pass1(lofs):
    @functools.partial(
        pl.kernel,
        out_type=jax.ShapeDtypeStruct((E * 16,), jnp.float32),
        mesh=_mesh,
        scratch_types=[
            pltpu.VMEM((CH,), jnp.int32),
            pltpu.VMEM((CH,), jnp.int32),
            pltpu.VMEM((CH,), jnp.int32),
            pltpu.VMEM((CH,), jnp.int32),
            pltpu.VMEM((CH, FBW), jnp.float32),
            pltpu.VMEM((CH, FBW), jnp.float32),
            pltpu.VMEM((CH, FBW), jnp.float32),
            pltpu.VMEM((FB, FBW), jnp.float32),
            pltpu.VMEM((CH * 16,), jnp.float32),
            pltpu.SemaphoreType.DMA,
            pltpu.SemaphoreType.DMA,
        ],
    )
    def _sc_pass1(xlxr_hbm, ee_hbm, src_hbm, dst_hbm, att_hbm,
                  part_out,
                  srcv, dstv, sidx, didx, xlr, xrr, eer, attv,
                  alanes, sem1, sem2):
        wid = lax.axis_index("s") * NC + lax.axis_index("c")

        pltpu.sync_copy(att_hbm, attv)

        def chunk_body(t, _):
            cid = wid + t * NW

            @pl.when(cid < NCHUNKS)
            def _():
                eb = cid * CH
                pltpu.sync_copy(src_hbm.at[pl.ds(eb, CH)], srcv)
                pltpu.sync_copy(dst_hbm.at[pl.ds(eb, CH)], dstv)

                for fb in range(FB):
                    off = fb * NPAD
                    offr = (FB + fb) * NPAD
                    for g in range(CH // 16):
                        sl = pl.ds(g * 16, 16)
                        sidx[sl] = srcv[sl] + off
                        didx[sl] = dstv[sl] + offr
                    cp1 = pltpu.async_copy(xlxr_hbm.at[sidx], xlr, sem1)
                    cp2 = pltpu.async_copy(xlxr_hbm.at[didx], xrr, sem2)
                    cp1.wait()
                    cp2.wait()
                    pltpu.sync_copy(
                        ee_hbm.at[pl.ds((lofs + fb) * E + eb, CH)], eer)

                    att_vs = [attv[fb, pl.ds(j * 16, 16)]
                              for j in range(FBW // 16)]

                    def edge_body(i, _, fb=fb, att_vs=att_vs):
                        acc = None
                        for j in range(FBW // 16):
                            sl = pl.ds(j * 16, 16)
                            mm = xlr[i, sl] + xrr[i, sl] + eer[i, sl]
                            mm = jnp.maximum(mm, 0.2 * mm)
                            contrib = mm * att_vs[j]
                            acc = contrib if acc is None else acc + contrib
                        osl = pl.ds(i * 16, 16)
                        if fb == 0:
                            alanes[osl] = acc
                        else:
                            alanes[osl] = alanes[osl] + acc
                        return 0
                    lax.fori_loop(0, CH, edge_body, 0)

                pltpu.sync_copy(alanes, part_out.at[pl.ds(eb * 16, CH * 16)])
            return 0

        lax.fori_loop(0, TMAX, chunk_body, 0)

    return _sc_pass1


def _tc_edge_w(part, dmat):
    rows = E * 16 // FBW
    bm = 800
    grid = (rows // bm,)

    def body(p_ref, d_ref, o_ref):
        a = jnp.dot(p_ref[...], d_ref[...], preferred_element_type=jnp.float32)
        o_ref[...] = jnp.exp(a)

    return pl.pallas_call(
        body,
        grid=grid,
        in_specs=[
            pl.BlockSpec((bm, FBW), lambda i: (i, 0)),
            pl.BlockSpec((FBW, FBW), lambda i: (0, 0)),
        ],
        out_specs=pl.BlockSpec((bm, FBW), lambda i: (i, 0)),
        out_shape=jax.ShapeDtypeStruct((rows, FBW), jnp.float32),
    )(part.reshape(rows, FBW), dmat)


@functools.partial(
    pl.kernel,
    out_type=[
        jax.ShapeDtypeStruct((NC, FB, NPAD, FBW), jnp.float32),
        jax.ShapeDtypeStruct((NC, NPAD), jnp.float32),
    ],
    mesh=_mesh,
    scratch_types=[
        pltpu.VMEM((2, CH), jnp.int32),
        pltpu.VMEM((2, CH), jnp.int32),
        pltpu.VMEM((2, CH), jnp.int32),
        pltpu.VMEM((2, CH * 16 // FBW, FBW), jnp.float32),
        pltpu.VMEM((CH,), jnp.float32),
        pltpu.VMEM((2, CH, FBW), jnp.float32),
        pltpu.VMEM((64, FBW), jnp.float32),
        pltpu.VMEM_SHARED((NPAD, FBW), jnp.float32),
        pltpu.VMEM_SHARED((NPAD,), jnp.float32),
        pltpu.SemaphoreType.DMA,
        pltpu.SemaphoreType.DMA,
        pltpu.SemaphoreType.DMA,
        pltpu.SemaphoreType.DMA,
    ],
)
def _sc_pass2(xl_hbm, src_hbm, dst_hbm, w_hbm,
              out_hbm, dpart_out,
              srcv2, dstv2, sidx2, wb2, wbufc, xlr2, zrow, acc, densh,
              sems0, sems1, semg0, semg1):
    cidx = lax.axis_index("c")
    sid = lax.axis_index("s")
    wid = sid * NC + cidx
    semss = [sems0, sems1]
    semgs = [semg0, semg1]

    def zrow_body(i, _):
        for j in range(FBW // 16):
            zrow[i, pl.ds(j * 16, 16)] = jnp.zeros((16,), jnp.float32)
        return 0
    lax.fori_loop(0, 64, zrow_body, 0)

    rows_per_sub = NPAD // NS

    for fb in range(FB):
        for k in range(rows_per_sub // 64):
            pltpu.sync_copy(zrow, acc.at[pl.ds(sid * rows_per_sub + k * 64, 64)])
        if fb == 0:
            for k in range(rows_per_sub // FBW):
                pltpu.sync_copy(
                    zrow.at[0],
                    densh.at[pl.ds(sid * rows_per_sub + k * FBW, FBW)])
        plsc.subcore_barrier()

        def fire(t, b, fb=fb):
            cid = wid + t * NW

            @pl.when(cid < NCHUNKS)
            def _():
                eb = cid * CH
                c1 = pltpu.async_copy(src_hbm.at[pl.ds(eb, CH)],
                                      srcv2.at[b], semss[b])
                c2 = pltpu.async_copy(dst_hbm.at[pl.ds(eb, CH)],
                                      dstv2.at[b], semss[b])
                c3 = pltpu.async_copy(
                    w_hbm.at[pl.ds(cid * (CH * 16 // FBW), CH * 16 // FBW)],
                    wb2.at[b], semss[b])
                c1.wait()
                c2.wait()
                c3.wait()
                off = fb * NPAD
                for g in range(CH // 16):
                    sl = pl.ds(g * 16, 16)
                    sidx2[b, sl] = srcv2[b, sl] + off
                pltpu.async_copy(xl_hbm.at[sidx2.at[b]], xlr2.at[b], semgs[b])

        def consume(t, b, fb=fb):
            cid = wid + t * NW

            @pl.when(cid < NCHUNKS)
            def _():
                pltpu.make_async_copy(xl_hbm.at[sidx2.at[b]], xlr2.at[b],
                                      semgs[b]).wait()

                if fb == 0:
                    iv16 = lax.iota(jnp.int32, 16)
                    for g in range(CH // 16):
                        wc = jnp.zeros((16,), jnp.float32)
                        for k in range(16):
                            v = wb2[b, 2 * g + k // 8,
                                    pl.ds((k % 8) * 16, 16)]
                            wc = jnp.where(iv16 == k, v[0], wc)
                        wbufc[pl.ds(g * 16, 16)] = wc
                    pltpu.sync_copy(wbufc, densh.at[dstv2.at[b]], add=True)

                def scale_body(g, _):
                    for k in range(16):
                        i = g * 16 + k
                        wsv = wb2[b, 2 * g + k // 8, pl.ds((k % 8) * 16, 16)]
                        for j in range(FBW // 16):
                            sl = pl.ds(j * 16, 16)
                            xlr2[b, i, sl] = xlr2[b, i, sl] * wsv
                    return 0
                lax.fori_loop(0, CH // 16, scale_body, 0)

                pltpu.sync_copy(xlr2.at[b], acc.at[dstv2.at[b]], add=True)

        fire(0, 0)

        def super_body(s, _):
            k0 = s * 2
            fire(k0 + 1, 1)
            consume(k0, 0)
            fire(k0 + 2, 0)
            consume(k0 + 1, 1)
            return 0

        lax.fori_loop(0, TMAX // 2, super_body, 0)
        plsc.subcore_barrier()

        for k in range(rows_per_sub // 64):
            base = sid * rows_per_sub + k * 64
            pltpu.sync_copy(acc.at[pl.ds(base, 64)],
                            out_hbm.at[cidx, fb, pl.ds(base, 64)])
        if fb == 0:
            pltpu.sync_copy(densh.at[pl.ds(sid * rows_per_sub, rows_per_sub)],
                            dpart_out.at[cidx, pl.ds(sid * rows_per_sub,
                                                     rows_per_sub)])
        plsc.subcore_barrier()


def _tc_combine(outp, dpart, bias2d):
    bn = 1024
    grid = (NPAD // bn, FB)

    def body(o_ref, d_ref, b_ref, h_ref):
        s = o_ref[0, 0] + o_ref[1, 0]
        den = jnp.sum(d_ref[...], axis=0)
        x = s / (den[:, None] + 1e-16) + b_ref[0:1]
        h_ref[...] = jnp.where(x > 0.0, x, jnp.exp(jnp.minimum(x, 0.0)) - 1.0)

    return pl.pallas_call(
        body,
        grid=grid,
        in_specs=[
            pl.BlockSpec((NC, 1, bn, FBW), lambda i, f: (0, f, i, 0)),
            pl.BlockSpec((NC, bn), lambda i, f: (0, i)),
            pl.BlockSpec((8, FBW), lambda i, f: (f, 0)),
        ],
        out_specs=pl.BlockSpec((bn, FBW), lambda i, f: (i, f)),
        out_shape=jax.ShapeDtypeStruct((NPAD, HID), jnp.float32),
    )(outp, dpart, bias2d)


def _tc_combine_mm(outp, dpart, bias2d, wcat, bcat2d):
    bm = 512
    nb = wcat.shape[1] // FBW
    grid = (NPAD // bm, nb)

    def body(o_ref, d_ref, b_ref, w_ref, bc_ref, x_ref, h_scr):
        f = pl.program_id(1)

        @pl.when(f == 0)
        def _():
            den = jnp.sum(d_ref[...], axis=0)[:, None]
            for fb in range(FB):
                s = o_ref[0, fb] + o_ref[1, fb]
                x = s / (den + 1e-16) + b_ref[fb * 8:fb * 8 + 1]
                h_scr[:, fb * FBW:(fb + 1) * FBW] = jnp.where(
                    x > 0.0, x, jnp.exp(jnp.minimum(x, 0.0)) - 1.0)

        x_ref[...] = (
            jnp.dot(h_scr[...], w_ref[...], preferred_element_type=jnp.float32)
            + bc_ref[0:1]
        )

    return pl.pallas_call(
        body,
        grid=grid,
        in_specs=[
            pl.BlockSpec((NC, FB, bm, FBW), lambda i, f: (0, 0, i, 0)),
            pl.BlockSpec((NC, bm), lambda i, f: (0, i)),
            pl.BlockSpec((FB * 8, FBW), lambda i, f: (0, 0)),
            pl.BlockSpec((HID, FBW), lambda i, f: (0, f)),
            pl.BlockSpec((8, FBW), lambda i, f: (f, 0)),
        ],
        out_specs=pl.BlockSpec((bm, FBW),
                               lambda i, f: (f * (NPAD // bm) + i, 0)),
        out_shape=jax.ShapeDtypeStruct((nb * NPAD, FBW), jnp.float32),
        scratch_shapes=[pltpu.VMEM((bm, HID), jnp.float32)],
    )(outp, dpart, bias2d, wcat, bcat2d)


def _tc_pool_mlp(h, batch2d, wf1, bf1, wf2, bf2):
    bp = 1024
    nsteps = NPAD // bp

    def body(h_ref, b_ref, w1_ref, b1_ref, w2_ref, b2_ref, o_ref,
             pooled, cnt):
        i = pl.program_id(0)

        @pl.when(i == 0)
        def _():
            pooled[...] = jnp.zeros_like(pooled)
            cnt[...] = jnp.zeros_like(cnt)

        oh = (b_ref[...] == lax.broadcasted_iota(jnp.int32, (bp, G), 1)
              ).astype(jnp.float32)
        pooled[...] += lax.dot_general(
            oh, h_ref[...], (((0,), (0,)), ((), ())),
            preferred_element_type=jnp.float32)
        cnt[...] += jnp.sum(oh, axis=0)[:, None]

        @pl.when(i == nsteps - 1)
        def _():
            c = jnp.maximum(cnt[:, 0:1], 1.0)
            pm = pooled[...] / c
            z1 = jnp.dot(pm, w1_ref[...], preferred_element_type=jnp.float32)
            z1 = jnp.maximum(z1 + b1_ref[...], 0.0)
            z = jnp.dot(z1, w2_ref[...], preferred_element_type=jnp.float32)
            z = z + b2_ref[...]
            mz = jnp.max(z, axis=1, keepdims=True)
            sz = z - mz
            o_ref[...] = sz - jnp.log(jnp.sum(jnp.exp(sz), axis=1,
                                              keepdims=True))

    return pl.pallas_call(
        body,
        grid=(nsteps,),
        in_specs=[
            pl.BlockSpec((bp, HID), lambda i: (i, 0)),
            pl.BlockSpec((bp, 1), lambda i: (i, 0)),
            pl.BlockSpec((HID, FC), lambda i: (0, 0)),
            pl.BlockSpec((1, FC), lambda i: (0, 0)),
            pl.BlockSpec((FC, C), lambda i: (0, 0)),
            pl.BlockSpec((1, C), lambda i: (0, 0)),
        ],
        out_specs=pl.BlockSpec((G, C), lambda i: (0, 0)),
        out_shape=jax.ShapeDtypeStruct((G, C), jnp.float32),
        scratch_shapes=[
            pltpu.VMEM((G, HID), jnp.float32),
            pltpu.VMEM((G, FBW), jnp.float32),
        ],
    )(h, batch2d, wf1, bf1, wf2, bf2)


def _rep_bias(b):
    nb = b.shape[0] // FBW
    return jnp.broadcast_to(b.reshape(nb, 1, FBW), (nb, 8, FBW)).reshape(
        nb * 8, FBW)


def kernel(x, edge_index, edge_attr, batch, params):
    src = edge_index[0]
    dst = edge_index[1]

    h = jnp.pad(x, ((0, NPAD - N), (0, 0)))
    batch2d = jnp.concatenate(
        [batch, jnp.full((NPAD - N,), G, jnp.int32)]).reshape(NPAD, 1)
    ii = jnp.arange(FBW, dtype=jnp.int32)
    dmat = (ii[:, None] // 16 == ii[None, :] // 16).astype(jnp.float32)

    zero_b = jnp.zeros((FB * 8, FBW), jnp.float32)
    ees = [_mm_fb(edge_attr, params[l][4], zero_b, 1600) for l in range(L)]

    wl, bl, wr, br = params[0][:4]
    xlxr = _mm_fb(h, jnp.concatenate([wl, wr], axis=1),
                  _rep_bias(jnp.concatenate([bl, br])), 512)

    _sc_pass1 = _make_sc_pass1(0)
    for l in range(L):
        att, bias = params[l][5], params[l][6]
        part = _sc_pass1(xlxr, ees[l], src, dst, att.reshape(FB, FBW))
        w2d = _tc_edge_w(part, dmat)
        outp, dpart = _sc_pass2(xlxr, src, dst, w2d)
        if l < L - 1:
            wln, bln, wrn, brn = params[l + 1][:4]
            xlxr = _tc_combine_mm(
                outp, dpart, _rep_bias(bias),
                jnp.concatenate([wln, wrn], axis=1),
                _rep_bias(jnp.concatenate([bln, brn])))
        else:
            h = _tc_combine(outp, dpart, _rep_bias(bias))

    wf1, bf1, wf2, bf2 = params[L]
    return _tc_pool_mlp(h, batch2d, wf1, bf1.reshape(1, FC),
                        wf2, bf2.reshape(1, C))
